# baseline (speedup 1.0000x reference)
import functools

import jax
import jax.numpy as jnp
from jax import lax
from jax.experimental import pallas as pl
from jax.experimental.pallas import tpu as pltpu
from jax.experimental.pallas import tpu_sc as plsc

NC = 2
NS = 16
NW = NC * NS
CHUNK = 128
EXPORT_CHUNK = 128
LANES = 16


NBUF = 4


def _sc_segment_sum(d_feat, cpw, npad, gather):
    rows_pt = npad // NS
    mesh = plsc.VectorSubcoreMesh(core_axis_name="c", subcore_axis_name="s")

    scratch = [
        pltpu.VMEM((cpw, CHUNK), jnp.int32),
        pltpu.VMEM((NBUF, CHUNK, d_feat), jnp.float32),
        pltpu.VMEM((EXPORT_CHUNK, d_feat), jnp.float32),
        pltpu.VMEM_SHARED((npad, d_feat), jnp.float32),
        pltpu.SemaphoreType.DMA,
        [pltpu.SemaphoreType.DMA] * NBUF,
        [pltpu.SemaphoreType.DMA] * NBUF,
    ]
    if gather:
        scratch = [pltpu.VMEM((cpw, CHUNK), jnp.int32)] + scratch

    def body(*refs):
        if gather:
            (table_hbm, src_hbm, dst_hbm, out_hbm,
             src_v, dst_v, rows_v, stage_v, acc_sh, sem, gsem, ssem) = refs
        else:
            (dst_hbm, out_hbm,
             dst_v, rows_v, stage_v, acc_sh, sem, gsem, ssem) = refs
        cid = lax.axis_index("c")
        sid = lax.axis_index("s")
        wid = cid * NS + sid

        zero16 = jnp.zeros((LANES,), jnp.float32)

        def zero_stage(i, carry):
            for j in range(d_feat // LANES):
                stage_v[i, pl.ds(j * LANES, LANES)] = zero16
            return carry
        lax.fori_loop(0, EXPORT_CHUNK, zero_stage, 0)

        if not gather:
            one16 = jnp.full((LANES,), 1.0, jnp.float32)

            def fill_ones(i, carry):
                for j in range(d_feat // LANES):
                    rows_v[0, i, pl.ds(j * LANES, LANES)] = one16
                return carry
            lax.fori_loop(0, CHUNK, fill_ones, 0)

        def zero_acc(i, carry):
            r0 = sid * rows_pt + i * EXPORT_CHUNK
            pltpu.sync_copy(stage_v, acc_sh.at[pl.ds(r0, EXPORT_CHUNK)])
            return carry
        lax.fori_loop(0, rows_pt // EXPORT_CHUNK, zero_acc, 0)

        pltpu.sync_copy(dst_hbm.at[wid], dst_v)
        if gather:
            pltpu.sync_copy(src_hbm.at[wid], src_v)

        plsc.subcore_barrier()

        ngroups = cpw // NBUF
        if gather:
            for b in range(NBUF):
                pltpu.async_copy(table_hbm.at[src_v.at[b]], rows_v.at[b],
                                 gsem[b])

            def edge_group(g, carry):
                for b in range(NBUF):
                    j = g * NBUF + b
                    pltpu.make_async_copy(table_hbm.at[src_v.at[j]],
                                          rows_v.at[b], gsem[b]).wait()
                    pltpu.sync_copy(rows_v.at[b], acc_sh.at[dst_v.at[j]],
                                    add=True)
                    pltpu.async_copy(table_hbm.at[src_v.at[j + NBUF]],
                                     rows_v.at[b], gsem[b])
                return carry
            lax.fori_loop(0, ngroups - 1, edge_group, 0)
            for b in range(NBUF):
                j = cpw - NBUF + b
                pltpu.make_async_copy(table_hbm.at[src_v.at[j]],
                                      rows_v.at[b], gsem[b]).wait()
                pltpu.sync_copy(rows_v.at[b], acc_sh.at[dst_v.at[j]],
                                add=True)
        else:
            ones = rows_v.at[0]
            for b in range(NBUF):
                pltpu.async_copy(ones, acc_sh.at[dst_v.at[b]], ssem[b],
                                 add=True)

            def edge_group(g, carry):
                for b in range(NBUF):
                    j = g * NBUF + b
                    pltpu.make_async_copy(ones, acc_sh.at[dst_v.at[j]],
                                          ssem[b]).wait()
                    pltpu.async_copy(ones, acc_sh.at[dst_v.at[j + NBUF]],
                                     ssem[b], add=True)
                return carry
            lax.fori_loop(0, ngroups - 1, edge_group, 0)
            for b in range(NBUF):
                j = cpw - NBUF + b
                pltpu.make_async_copy(ones, acc_sh.at[dst_v.at[j]],
                                      ssem[b]).wait()

        plsc.subcore_barrier()

        def export(i, carry):
            r0 = sid * rows_pt + i * EXPORT_CHUNK
            pltpu.sync_copy(acc_sh.at[pl.ds(r0, EXPORT_CHUNK)], stage_v)
            pltpu.sync_copy(stage_v, out_hbm.at[cid, pl.ds(r0, EXPORT_CHUNK)])
            return carry
        lax.fori_loop(0, rows_pt // EXPORT_CHUNK, export, 0)

    return functools.partial(
        pl.kernel,
        out_type=jax.ShapeDtypeStruct((NC, npad, d_feat), jnp.float32),
        mesh=mesh,
        scratch_types=scratch,
        compiler_params=pltpu.CompilerParams(use_tc_tiling_on_sc=False),
    )(body)


BLK = 256
_DOT = dict(preferred_element_type=jnp.float32, precision=lax.Precision.HIGHEST)


def _tc_scale_matmul(x_ref, w1_ref, d0_ref, d1_ref, outa_ref, outb_ref):
    dis = lax.rsqrt(d0_ref[:, :1] + d1_ref[:, :1] + 1.0)
    h = jnp.dot(x_ref[:, :], w1_ref[:, :], **_DOT) * dis
    half = h.shape[1] // 2
    outa_ref[:, :] = h[:, :half]
    outb_ref[:, :] = h[:, half:]


def _tc_mid(ha_ref, hb_ref, pa0_ref, pa1_ref, pb0_ref, pb1_ref,
            d0_ref, d1_ref, b1_ref, w2_ref, wo_ref, out_ref):
    dis = lax.rsqrt(d0_ref[:, :1] + d1_ref[:, :1] + 1.0)
    agg_a = pa0_ref[:, :] + pa1_ref[:, :] + ha_ref[:, :]
    agg_b = pb0_ref[:, :] + pb1_ref[:, :] + hb_ref[:, :]
    pre = dis * jnp.concatenate([agg_a, agg_b], axis=1) + b1_ref[:, :]
    z = jnp.maximum(pre, 0.0)
    h2 = jnp.dot(jnp.dot(z, w2_ref[:, :], **_DOT), wo_ref[:, :], **_DOT)
    out_ref[:, :] = h2 * dis


def _tc_head(q0_ref, q1_ref, h2p_ref, d0_ref, d1_ref, b2_ref, wo_ref, bo_ref,
             out_ref, *, cpad, nclass):
    dis = lax.rsqrt(d0_ref[:, :1] + d1_ref[:, :1] + 1.0)
    bias = jnp.dot(b2_ref[:, :], wo_ref[:, :], **_DOT) + bo_ref[:, :]
    u = dis * (q0_ref[:, :] + q1_ref[:, :] + h2p_ref[:, :]) + bias
    col = lax.broadcasted_iota(jnp.int32, (BLK, cpad), 1)
    valid = col < nclass
    um = jnp.where(valid, u, jnp.float32(-1e30))
    m = jnp.max(um, axis=1, keepdims=True)
    e = jnp.where(valid, jnp.exp(um - m), 0.0)
    s = jnp.sum(e, axis=1, keepdims=True)
    out_ref[:, :] = u - m - jnp.log(s)


def _row_spec(width):
    return pl.BlockSpec((BLK, width), lambda i: (i, 0))


def _full_spec(shape):
    return pl.BlockSpec(shape, lambda i: (0,) * len(shape))


def kernel(x, edge_index, W1, b1, W2, b2, Wo, bo):
    n_nodes, d_feat = x.shape
    n_hid = W1.shape[1]
    n_class = Wo.shape[1]
    cpad = ((n_class + LANES - 1) // LANES) * LANES
    align = NS * EXPORT_CHUNK
    npad = ((n_nodes + 1 + align - 1) // align) * align
    n_edges = edge_index.shape[1]
    cpw = (n_edges + NW * CHUNK - 1) // (NW * CHUNK)
    cpw = max(((cpw + NBUF - 1) // NBUF) * NBUF, 2 * NBUF)
    e_pad = NW * CHUNK * cpw

    src = edge_index[0].astype(jnp.int32)
    dst = edge_index[1].astype(jnp.int32)
    fill = jnp.full((e_pad - n_edges,), n_nodes, jnp.int32)
    src_slab = jnp.concatenate([src, fill]).reshape(NW, cpw, CHUNK)
    dst_slab = jnp.concatenate([dst, fill]).reshape(NW, cpw, CHUNK)

    x_pad = jnp.pad(x, ((0, npad - n_nodes), (0, 0)))
    wo_pad = jnp.pad(Wo, ((0, 0), (0, cpad - n_class)))
    bo_pad = jnp.pad(bo, (0, cpad - n_class)).reshape(1, cpad)
    b1_2d = b1.reshape(1, n_hid)
    b2_2d = b2.reshape(1, n_hid)

    grid = (npad // BLK,)

    degp = _sc_segment_sum(LANES, cpw, npad, gather=False)(dst_slab)
    d0, d1 = degp[0], degp[1]

    half = n_hid // 2
    ha, hb = pl.pallas_call(
        _tc_scale_matmul,
        grid=grid,
        in_specs=[_row_spec(d_feat), _full_spec((d_feat, n_hid)),
                  _row_spec(LANES), _row_spec(LANES)],
        out_specs=[_row_spec(half), _row_spec(half)],
        out_shape=[jax.ShapeDtypeStruct((npad, half), jnp.float32),
                   jax.ShapeDtypeStruct((npad, half), jnp.float32)],
    )(x_pad, W1, d0, d1)

    pa = _sc_segment_sum(half, cpw, npad, gather=True)(ha, src_slab, dst_slab)
    pb = _sc_segment_sum(half, cpw, npad, gather=True)(hb, src_slab, dst_slab)

    h2p = pl.pallas_call(
        _tc_mid,
        grid=grid,
        in_specs=[_row_spec(half), _row_spec(half),
                  _row_spec(half), _row_spec(half),
                  _row_spec(half), _row_spec(half),
                  _row_spec(LANES), _row_spec(LANES), _full_spec((1, n_hid)),
                  _full_spec((n_hid, n_hid)), _full_spec((n_hid, cpad))],
        out_specs=_row_spec(cpad),
        out_shape=jax.ShapeDtypeStruct((npad, cpad), jnp.float32),
    )(ha, hb, pa[0], pa[1], pb[0], pb[1], d0, d1, b1_2d, W2, wo_pad)

    agg2p = _sc_segment_sum(cpad, cpw, npad, gather=True)(h2p, src_slab, dst_slab)

    out = pl.pallas_call(
        functools.partial(_tc_head, cpad=cpad, nclass=n_class),
        grid=grid,
        in_specs=[_row_spec(cpad), _row_spec(cpad), _row_spec(cpad),
                  _row_spec(LANES), _row_spec(LANES), _full_spec((1, n_hid)),
                  _full_spec((n_hid, cpad)), _full_spec((1, cpad))],
        out_specs=_row_spec(cpad),
        out_shape=jax.ShapeDtypeStruct((npad, cpad), jnp.float32),
    )(agg2p[0], agg2p[1], h2p, d0, d1, b2_2d, wo_pad, bo_pad)

    return out[:n_nodes, :n_class]


# speedup vs baseline: 1.3460x; 1.3460x over previous
import functools

import jax
import jax.numpy as jnp
from jax import lax
from jax.experimental import pallas as pl
from jax.experimental.pallas import tpu as pltpu
from jax.experimental.pallas import tpu_sc as plsc

NC = 2
NS = 16
CHUNK = 128
EXPORT_CHUNK = 128
LANES = 16


def _sc_edge_kernel(d_feat, cpw, npad, mode):
    rows_pt = npad // NS
    cpw_eff = 2 * cpw if mode == "split" else cpw
    mesh = plsc.VectorSubcoreMesh(core_axis_name="c", subcore_axis_name="s")

    scratch = [
        pltpu.VMEM((cpw_eff, CHUNK), jnp.int32),
        pltpu.VMEM((CHUNK, d_feat), jnp.float32),
        pltpu.VMEM((EXPORT_CHUNK, d_feat), jnp.float32),
        pltpu.VMEM_SHARED((npad, d_feat), jnp.float32),
        pltpu.SemaphoreType.DMA,
    ]
    if mode != "deg":
        scratch = [pltpu.VMEM((cpw_eff, CHUNK), jnp.int32)] + scratch

    def body(*refs):
        if mode != "deg":
            (table_hbm, src_hbm, dst_hbm, out_hbm,
             src_v, dst_v, rows_v, stage_v, acc_sh, sem) = refs
        else:
            (dst_hbm, out_hbm,
             dst_v, rows_v, stage_v, acc_sh, sem) = refs
        cid = lax.axis_index("c")
        sid = lax.axis_index("s")

        zero16 = jnp.zeros((LANES,), jnp.float32)

        def zero_stage(i, carry):
            for j in range(d_feat // LANES):
                stage_v[i, pl.ds(j * LANES, LANES)] = zero16
            return carry
        lax.fori_loop(0, EXPORT_CHUNK, zero_stage, 0)

        if mode == "deg":
            one16 = jnp.full((LANES,), 1.0, jnp.float32)

            def fill_ones(i, carry):
                for j in range(d_feat // LANES):
                    rows_v[i, pl.ds(j * LANES, LANES)] = one16
                return carry
            lax.fori_loop(0, CHUNK, fill_ones, 0)

        def zero_acc(i, carry):
            r0 = sid * rows_pt + i * EXPORT_CHUNK
            pltpu.sync_copy(stage_v, acc_sh.at[pl.ds(r0, EXPORT_CHUNK)])
            return carry
        lax.fori_loop(0, rows_pt // EXPORT_CHUNK, zero_acc, 0)

        if mode == "split":
            pltpu.sync_copy(dst_hbm.at[sid], dst_v)
            pltpu.sync_copy(src_hbm.at[cid, sid], src_v)
        else:
            col = cid * cpw
            pltpu.sync_copy(dst_hbm.at[sid, pl.ds(col, cpw)], dst_v)
            if mode == "partial":
                pltpu.sync_copy(src_hbm.at[0, sid, pl.ds(col, cpw)], src_v)

        plsc.subcore_barrier()

        def edge_body(j, carry):
            if mode != "deg":
                pltpu.async_copy(table_hbm.at[src_v.at[j]], rows_v, sem).wait()
            pltpu.sync_copy(rows_v, acc_sh.at[dst_v.at[j]], add=True)
            return carry
        lax.fori_loop(0, cpw_eff, edge_body, 0)

        plsc.subcore_barrier()

        def export(i, carry):
            r0 = sid * rows_pt + i * EXPORT_CHUNK
            pltpu.sync_copy(acc_sh.at[pl.ds(r0, EXPORT_CHUNK)], stage_v)
            pltpu.sync_copy(stage_v, out_hbm.at[cid, pl.ds(r0, EXPORT_CHUNK)])
            return carry
        lax.fori_loop(0, rows_pt // EXPORT_CHUNK, export, 0)

    return functools.partial(
        pl.kernel,
        out_type=jax.ShapeDtypeStruct((NC, npad, d_feat), jnp.float32),
        mesh=mesh,
        scratch_types=scratch,
        compiler_params=pltpu.CompilerParams(use_tc_tiling_on_sc=False),
    )(body)


BLK = 256
_DOT = dict(preferred_element_type=jnp.float32, precision=lax.Precision.HIGHEST)


def _tc_scale_matmul(x_ref, w1_ref, d0_ref, d1_ref, out_ref):
    dis = lax.rsqrt(d0_ref[:, :1] + d1_ref[:, :1] + 1.0)
    h = jnp.dot(x_ref[:, :], w1_ref[:, :], **_DOT) * dis
    half = h.shape[1] // 2
    out_ref[0, :, :] = h[:, :half]
    out_ref[1, :, :] = h[:, half:]


def _tc_mid(ha_ref, hb_ref, ga_ref, gb_ref,
            d0_ref, d1_ref, b1_ref, w2_ref, wo_ref, out_ref):
    dis = lax.rsqrt(d0_ref[:, :1] + d1_ref[:, :1] + 1.0)
    agg = jnp.concatenate([ga_ref[:, :] + ha_ref[:, :],
                           gb_ref[:, :] + hb_ref[:, :]], axis=1)
    z = jnp.maximum(dis * agg + b1_ref[:, :], 0.0)
    h2 = jnp.dot(jnp.dot(z, w2_ref[:, :], **_DOT), wo_ref[:, :], **_DOT)
    out_ref[:, :] = h2 * dis


def _tc_head(q0_ref, q1_ref, h2p_ref, d0_ref, d1_ref, b2_ref, wo_ref, bo_ref,
             out_ref, *, cpad, nclass):
    dis = lax.rsqrt(d0_ref[:, :1] + d1_ref[:, :1] + 1.0)
    bias = jnp.dot(b2_ref[:, :], wo_ref[:, :], **_DOT) + bo_ref[:, :]
    u = dis * (q0_ref[:, :] + q1_ref[:, :] + h2p_ref[:, :]) + bias
    col = lax.broadcasted_iota(jnp.int32, (BLK, cpad), 1)
    valid = col < nclass
    um = jnp.where(valid, u, jnp.float32(-1e30))
    m = jnp.max(um, axis=1, keepdims=True)
    e = jnp.where(valid, jnp.exp(um - m), 0.0)
    s = jnp.sum(e, axis=1, keepdims=True)
    out_ref[:, :] = u - m - jnp.log(s)


def _row_spec(width):
    return pl.BlockSpec((BLK, width), lambda i: (i, 0))


def _full_spec(shape):
    return pl.BlockSpec(shape, lambda i: (0,) * len(shape))


def kernel(x, edge_index, W1, b1, W2, b2, Wo, bo):
    n_nodes, d_feat = x.shape
    n_hid = W1.shape[1]
    half = n_hid // 2
    n_class = Wo.shape[1]
    cpad = ((n_class + LANES - 1) // LANES) * LANES
    align = NS * EXPORT_CHUNK
    npad = ((n_nodes + 1 + align - 1) // align) * align
    n_edges = edge_index.shape[1]
    cpw = (n_edges + NC * NS * CHUNK - 1) // (NC * NS * CHUNK)
    e_pad = NC * NS * CHUNK * cpw

    src = edge_index[0].astype(jnp.int32)
    dst = edge_index[1].astype(jnp.int32)
    fill = jnp.full((e_pad - n_edges,), n_nodes, jnp.int32)
    src_flat = jnp.concatenate([src, fill]).reshape(NS, NC * cpw, CHUNK)
    dst_slab = jnp.concatenate([dst, fill]).reshape(NS, NC * cpw, CHUNK)
    src_slab = jnp.stack([src_flat, src_flat + npad])

    x_pad = jnp.pad(x, ((0, npad - n_nodes), (0, 0)))
    wo_pad = jnp.pad(Wo, ((0, 0), (0, cpad - n_class)))
    bo_pad = jnp.pad(bo, (0, cpad - n_class)).reshape(1, cpad)
    b1_2d = b1.reshape(1, n_hid)
    b2_2d = b2.reshape(1, n_hid)

    grid = (npad // BLK,)

    degp = _sc_edge_kernel(LANES, cpw, npad, "deg")(dst_slab)
    d0, d1 = degp[0], degp[1]

    hs = pl.pallas_call(
        _tc_scale_matmul,
        grid=grid,
        in_specs=[_row_spec(d_feat), _full_spec((d_feat, n_hid)),
                  _row_spec(LANES), _row_spec(LANES)],
        out_specs=pl.BlockSpec((NC, BLK, half), lambda i: (0, i, 0)),
        out_shape=jax.ShapeDtypeStruct((NC, npad, half), jnp.float32),
    )(x_pad, W1, d0, d1)

    table = hs.reshape(NC * npad, half)
    agg = _sc_edge_kernel(half, cpw, npad, "split")(table, src_slab, dst_slab)

    h2p = pl.pallas_call(
        _tc_mid,
        grid=grid,
        in_specs=[_row_spec(half), _row_spec(half),
                  _row_spec(half), _row_spec(half),
                  _row_spec(LANES), _row_spec(LANES), _full_spec((1, n_hid)),
                  _full_spec((n_hid, n_hid)), _full_spec((n_hid, cpad))],
        out_specs=_row_spec(cpad),
        out_shape=jax.ShapeDtypeStruct((npad, cpad), jnp.float32),
    )(hs[0], hs[1], agg[0], agg[1], d0, d1, b1_2d, W2, wo_pad)

    agg2p = _sc_edge_kernel(cpad, cpw, npad, "partial")(h2p, src_slab, dst_slab)

    out = pl.pallas_call(
        functools.partial(_tc_head, cpad=cpad, nclass=n_class),
        grid=grid,
        in_specs=[_row_spec(cpad), _row_spec(cpad), _row_spec(cpad),
                  _row_spec(LANES), _row_spec(LANES), _full_spec((1, n_hid)),
                  _full_spec((n_hid, cpad)), _full_spec((1, cpad))],
        out_specs=_row_spec(cpad),
        out_shape=jax.ShapeDtypeStruct((npad, cpad), jnp.float32),
    )(agg2p[0], agg2p[1], h2p, d0, d1, b2_2d, wo_pad, bo_pad)

    return out[:n_nodes, :n_class]
